# baseline (speedup 1.0000x reference)
import functools

import jax
import jax.numpy as jnp
from jax import lax
from jax.experimental import pallas as pl
from jax.experimental.pallas import tpu as pltpu
from jax.experimental.pallas import tpu_sc as plsc

_EMB = 64
_ARITY = 6
_RELW = 896

_NC, _NS = 2, 16
_NW = _NC * _NS
_BPW = 128
_HIGH = jax.lax.Precision.HIGHEST


def _sc_gather(EP, relT, eidx, ridx):
    n_r = ridx.shape[0]
    mesh = plsc.VectorSubcoreMesh(core_axis_name="c", subcore_axis_name="s")

    @functools.partial(
        pl.kernel,
        mesh=mesh,
        out_type=(
            jax.ShapeDtypeStruct((n_r, _ARITY * 128), jnp.float32),
            jax.ShapeDtypeStruct((n_r, _RELW), jnp.float32),
        ),
        scratch_types=[
            pltpu.VMEM((_ARITY * _BPW,), jnp.int32),
            pltpu.VMEM((_BPW,), jnp.int32),
            pltpu.VMEM((2, _BPW, 128), jnp.float32),
            pltpu.VMEM((_BPW // 2, _RELW), jnp.float32),
            pltpu.SemaphoreType.DMA,
            pltpu.SemaphoreType.DMA,
            pltpu.SemaphoreType.DMA,
        ],
    )
    def k(EP_hbm, relT_hbm, eidx_hbm, ridx_hbm, emb_out, rel_out,
          eidx_v, ridx_v, emb_v, rel_v, sem0, sem1, sem_r):
        wid = lax.axis_index("s") * _NC + lax.axis_index("c")
        be = wid * _ARITY * _BPW
        br = wid * _BPW
        hb = _BPW // 2
        pltpu.sync_copy(eidx_hbm.at[pl.ds(be, _ARITY * _BPW)], eidx_v)
        pltpu.sync_copy(ridx_hbm.at[pl.ds(br, _BPW)], ridx_v)
        sems = (sem0, sem1)

        cp_r = pltpu.async_copy(relT_hbm.at[ridx_v.at[pl.ds(0, hb)]],
                                rel_v, sem_r)
        cps = [pltpu.async_copy(EP_hbm.at[eidx_v.at[pl.ds(a * _BPW, _BPW)]],
                                emb_v.at[a % 2], sems[a % 2])
               for a in range(2)]
        for a in range(_ARITY):
            cps[a % 2].wait()
            pltpu.sync_copy(
                emb_v.at[a % 2],
                emb_out.at[pl.ds(br, _BPW), pl.ds(a * 128, 128)])
            if a + 2 < _ARITY:
                cps[a % 2] = pltpu.async_copy(
                    EP_hbm.at[eidx_v.at[pl.ds((a + 2) * _BPW, _BPW)]],
                    emb_v.at[a % 2], sems[a % 2])
            if a == 0:
                cp_r.wait()
                pltpu.sync_copy(rel_v, rel_out.at[pl.ds(br, hb)])
                cp_r = pltpu.async_copy(relT_hbm.at[ridx_v.at[pl.ds(hb, hb)]],
                                        rel_v, sem_r)
        cp_r.wait()
        pltpu.sync_copy(rel_v, rel_out.at[pl.ds(br + hb, hb)])

    return k(EP, relT, eidx, ridx)


def _tc_body(emb_ref, rel_ref, hm_ref, out_ref):
    f32 = jnp.float32
    embp = emb_ref[...]
    rel = rel_ref[...]
    hm = hm_ref[...]
    parts = []
    for a_ in range(_ARITY):
        ch = embp[:, a_ * 128 : (a_ + 1) * 128]
        rot = jnp.concatenate([ch[:, 64:], ch[:, :64]], axis=1)
        parts.append(jnp.where(hm[:, a_ : a_ + 1] > 0.5, rot, ch))
    emb = jnp.concatenate(parts, axis=1)
    prod = emb * rel[:, : _ARITY * 128]
    tail = rel[:, _ARITY * 128 :]

    r = lax.broadcasted_iota(jnp.int32, (_ARITY * 128, 128), 0)
    c = lax.broadcasted_iota(jnp.int32, (_ARITY * 128, 128), 1)
    a = r // 128
    off = r % 128
    v = c // 8
    veven = v % 2 == 0
    active = (veven & (2 * a <= v)) | (~veven & (2 * a >= v + 1))
    M = jnp.where((off < 64) & (off % 8 == c % 8) & active & (v < 11),
                  1.0, 0.0).astype(f32)
    r2 = lax.broadcasted_iota(jnp.int32, (128, 128), 0)
    c2 = lax.broadcasted_iota(jnp.int32, (128, 128), 1)
    Mb = jnp.where((r2 < 8) & (r2 == c2 % 8) & (c2 < 88), 1.0, 0.0).astype(f32)
    x = jnp.tanh(jax.lax.dot(prod, M, precision=_HIGH,
                             preferred_element_type=f32)
                 + jax.lax.dot(tail, Mb, precision=_HIGH,
                               preferred_element_type=f32))

    W = jnp.where((r2 // 8 == c2) & (c2 < 11), 1.0, 0.0).astype(f32)
    s = jax.lax.dot(x, W, precision=_HIGH, preferred_element_type=f32)
    s = s + jnp.where(c2[:1, :] == 11, 1.0, 0.0).astype(f32)

    G = jnp.where((c2 % 2 == 0) & (r2 == 8 + c2 // 2) & (c2 < 12), 1.0,
                  0.0).astype(f32)
    wi = jax.lax.dot(tail, G, precision=_HIGH, preferred_element_type=f32)

    srot = jnp.concatenate([s[:, 1:], s[:, :1]], axis=1)
    out_ref[...] = jnp.sum(s * srot * wi, axis=1)


def _tc_compute(emb, rel, hm):
    B = rel.shape[0]
    BT = 512
    return pl.pallas_call(
        _tc_body,
        grid=(B // BT,),
        in_specs=[
            pl.BlockSpec((BT, _ARITY * 128), lambda i: (i, 0)),
            pl.BlockSpec((BT, _RELW), lambda i: (i, 0)),
            pl.BlockSpec((BT, 128), lambda i: (i, 0)),
        ],
        out_specs=pl.BlockSpec((BT,), lambda i: (i,)),
        out_shape=jax.ShapeDtypeStruct((B,), jnp.float32),
    )(emb, rel, hm)


def kernel(r_idx, e1_idx, e2_idx, e3_idx, e4_idx, e5_idx, e6_idx,
           E_w, R_w, R_bias_w, Rw0, Rw1, Rw2, Rw3, Rw4, Rw5):
    B = r_idx.shape[0]
    nrel = R_w.shape[0]
    E2 = E_w.reshape(E_w.shape[0] // 2, 2 * _EMB)
    z64 = jnp.zeros((nrel, 64), jnp.float32)
    chunks = []
    for a_ in range(_ARITY):
        chunks += [R_w[:, a_ * 64 : (a_ + 1) * 64], z64]
    chunks += [R_bias_w, Rw0, Rw1, Rw2, Rw3, Rw4, Rw5,
               jnp.zeros((nrel, 114), jnp.float32)]
    relT = jnp.concatenate(chunks, axis=1)
    eidx = jnp.stack(
        [e1_idx, e2_idx, e3_idx, e4_idx, e5_idx, e6_idx], axis=0
    ).reshape(_ARITY, _NW, _BPW).transpose(1, 0, 2).reshape(-1).astype(jnp.int32)
    hm = jnp.pad((eidx % 2).reshape(_NW, _ARITY, _BPW).transpose(0, 2, 1)
                 .reshape(B, _ARITY).astype(jnp.float32),
                 ((0, 0), (0, 128 - _ARITY)))
    emb, rel = _sc_gather(E2, relT, eidx // 2, r_idx.astype(jnp.int32))
    return _tc_compute(emb, rel, hm)


# speedup vs baseline: 1.1457x; 1.1457x over previous
import functools

import jax
import jax.numpy as jnp
from jax import lax
from jax.experimental import pallas as pl
from jax.experimental.pallas import tpu as pltpu
from jax.experimental.pallas import tpu_sc as plsc

_EMB = 64
_ARITY = 6
_RELW = 896

_NC, _NS = 2, 16
_NW = _NC * _NS
_BPW = 128
_HIGH = jax.lax.Precision.DEFAULT


def _sc_gather(EP, relT, eidx, ridx):
    n_r = ridx.shape[0]
    mesh = plsc.VectorSubcoreMesh(core_axis_name="c", subcore_axis_name="s")

    @functools.partial(
        pl.kernel,
        mesh=mesh,
        out_type=(
            jax.ShapeDtypeStruct((n_r, _ARITY * 128), jnp.float32),
            jax.ShapeDtypeStruct((n_r, _RELW), jnp.float32),
        ),
        scratch_types=[
            pltpu.VMEM((_ARITY * _BPW,), jnp.int32),
            pltpu.VMEM((_BPW,), jnp.int32),
            pltpu.VMEM((2, _BPW, 128), jnp.float32),
            pltpu.VMEM((_BPW // 2, _RELW), jnp.float32),
            pltpu.SemaphoreType.DMA,
            pltpu.SemaphoreType.DMA,
            pltpu.SemaphoreType.DMA,
        ],
    )
    def k(EP_hbm, relT_hbm, eidx_hbm, ridx_hbm, emb_out, rel_out,
          eidx_v, ridx_v, emb_v, rel_v, sem0, sem1, sem_r):
        wid = lax.axis_index("s") * _NC + lax.axis_index("c")
        be = wid * _ARITY * _BPW
        br = wid * _BPW
        hb = _BPW // 2
        pltpu.sync_copy(eidx_hbm.at[pl.ds(be, _ARITY * _BPW)], eidx_v)
        pltpu.sync_copy(ridx_hbm.at[pl.ds(br, _BPW)], ridx_v)
        sems = (sem0, sem1)

        cp_r = pltpu.async_copy(relT_hbm.at[ridx_v.at[pl.ds(0, hb)]],
                                rel_v, sem_r)
        cps = [pltpu.async_copy(EP_hbm.at[eidx_v.at[pl.ds(a * _BPW, _BPW)]],
                                emb_v.at[a % 2], sems[a % 2])
               for a in range(2)]
        for a in range(_ARITY):
            cps[a % 2].wait()
            pltpu.sync_copy(
                emb_v.at[a % 2],
                emb_out.at[pl.ds(br, _BPW), pl.ds(a * 128, 128)])
            if a + 2 < _ARITY:
                cps[a % 2] = pltpu.async_copy(
                    EP_hbm.at[eidx_v.at[pl.ds((a + 2) * _BPW, _BPW)]],
                    emb_v.at[a % 2], sems[a % 2])
            if a == 0:
                cp_r.wait()
                pltpu.sync_copy(rel_v, rel_out.at[pl.ds(br, hb)])
                cp_r = pltpu.async_copy(relT_hbm.at[ridx_v.at[pl.ds(hb, hb)]],
                                        rel_v, sem_r)
        cp_r.wait()
        pltpu.sync_copy(rel_v, rel_out.at[pl.ds(br + hb, hb)])

    return k(EP, relT, eidx, ridx)


def _tc_body(emb_ref, rel_ref, out_ref):
    f32 = jnp.float32
    emb = emb_ref[...]
    rel = rel_ref[...]
    prod = emb * rel[:, : _ARITY * 128]
    tail = rel[:, _ARITY * 128 :]

    r = lax.broadcasted_iota(jnp.int32, (_ARITY * 128, 128), 0)
    c = lax.broadcasted_iota(jnp.int32, (_ARITY * 128, 128), 1)
    a = r // 128
    off = r % 128
    v = c // 8
    veven = v % 2 == 0
    active = (veven & (2 * a <= v)) | (~veven & (2 * a >= v + 1))
    M = jnp.where((off < 64) & (off % 8 == c % 8) & active & (v < 11),
                  1.0, 0.0).astype(f32)
    r2 = lax.broadcasted_iota(jnp.int32, (128, 128), 0)
    c2 = lax.broadcasted_iota(jnp.int32, (128, 128), 1)
    Mb = jnp.where((r2 < 8) & (r2 == c2 % 8) & (c2 < 88), 1.0, 0.0).astype(f32)
    x = jnp.tanh(jax.lax.dot(prod, M, precision=_HIGH,
                             preferred_element_type=f32)
                 + jax.lax.dot(tail, Mb, precision=_HIGH,
                               preferred_element_type=f32))

    W = jnp.where((r2 // 8 == c2) & (c2 < 11), 1.0, 0.0).astype(f32)
    s = jax.lax.dot(x, W, precision=_HIGH, preferred_element_type=f32)
    s = s + jnp.where(c2[:1, :] == 11, 1.0, 0.0).astype(f32)

    G = jnp.where((c2 % 2 == 0) & (r2 == 8 + c2 // 2) & (c2 < 12), 1.0,
                  0.0).astype(f32)
    wi = jax.lax.dot(tail, G, precision=_HIGH, preferred_element_type=f32)

    srot = jnp.concatenate([s[:, 1:], s[:, :1]], axis=1)
    out_ref[...] = jnp.sum(s * srot * wi, axis=1)


def _tc_compute(emb, rel):
    B = rel.shape[0]
    BT = 1024
    return pl.pallas_call(
        _tc_body,
        grid=(B // BT,),
        in_specs=[
            pl.BlockSpec((BT, _ARITY * 128), lambda i: (i, 0)),
            pl.BlockSpec((BT, _RELW), lambda i: (i, 0)),
        ],
        out_specs=pl.BlockSpec((BT,), lambda i: (i,)),
        out_shape=jax.ShapeDtypeStruct((B,), jnp.float32),
    )(emb, rel)


def kernel(r_idx, e1_idx, e2_idx, e3_idx, e4_idx, e5_idx, e6_idx,
           E_w, R_w, R_bias_w, Rw0, Rw1, Rw2, Rw3, Rw4, Rw5):
    B = r_idx.shape[0]
    nrel = R_w.shape[0]
    EP = jnp.concatenate([E_w, jnp.zeros_like(E_w)], axis=1)
    z64 = jnp.zeros((nrel, 64), jnp.float32)
    chunks = []
    for a_ in range(_ARITY):
        chunks += [R_w[:, a_ * 64 : (a_ + 1) * 64], z64]
    chunks += [R_bias_w, Rw0, Rw1, Rw2, Rw3, Rw4, Rw5,
               jnp.zeros((nrel, 114), jnp.float32)]
    relT = jnp.concatenate(chunks, axis=1)
    eidx = jnp.stack(
        [e1_idx, e2_idx, e3_idx, e4_idx, e5_idx, e6_idx], axis=0
    ).reshape(_ARITY, _NW, _BPW).transpose(1, 0, 2).reshape(-1).astype(jnp.int32)
    emb, rel = _sc_gather(EP, relT, eidx, r_idx.astype(jnp.int32))
    return _tc_compute(emb, rel)
